# baseline (speedup 1.0000x reference)
import functools

import jax
import jax.numpy as jnp
from jax import lax
from jax.experimental import pallas as pl
from jax.experimental.pallas import tpu as pltpu
from jax.experimental.pallas import tpu_sc as plsc

N = 1271
D = 768
NP = 1280
NC = 2
NS = 16
CH = 3200
EP = NC * NS * CH
IDX_W = 128
N_STREAMS = CH // IDX_W
SLICE = NP * NP // NS
ZCH = 12800
NEG = -3.0e38


def _sc_counts_body(row_hbm, col_hbm, out_hbm,
                    a_sh, row_v, col_v, idx_v, ones_v, zero_v, sem):
    cid = lax.axis_index("c")
    sid = lax.axis_index("s")

    zeros16 = jnp.zeros((16,), jnp.float32)
    ones16 = jnp.ones((16,), jnp.float32)

    def fill_zero(i, _):
        zero_v[pl.ds(i * 16, 16)] = zeros16
        return 0
    lax.fori_loop(0, ZCH // 16, fill_zero, 0)
    for k in range(IDX_W // 16):
        ones_v[pl.ds(k * 16, 16)] = ones16

    for k in range(SLICE // ZCH):
        pltpu.sync_copy(zero_v, a_sh.at[pl.ds(sid * SLICE + k * ZCH, ZCH)])
    plsc.subcore_barrier()

    base = cid * (NS * CH) + sid * CH
    pltpu.sync_copy(row_hbm.at[pl.ds(base, CH)], row_v)
    pltpu.sync_copy(col_hbm.at[pl.ds(base, CH)], col_v)

    def compute_idx(j, _):
        for k in range(8):
            off = j * IDX_W + k * 16
            r = row_v[pl.ds(off, 16)]
            c = col_v[pl.ds(off, 16)]
            t = r >> 7
            idx_v[j, pl.ds(k * 16, 16)] = (
                (t << 17) + (t << 15) + (c << 7) + (r & 127))
        return 0
    lax.fori_loop(0, N_STREAMS, compute_idx, 0)

    descs = []
    for j in range(N_STREAMS):
        descs.append(
            pltpu.async_copy(ones_v, a_sh.at[idx_v.at[j]], sem, add=True))
    for d in descs:
        d.wait()
    plsc.subcore_barrier()

    pltpu.sync_copy(a_sh.at[pl.ds(sid * SLICE, SLICE)],
                    out_hbm.at[cid, pl.ds(sid * SLICE, SLICE)])


@jax.jit
def _sc_counts(row, col):
    mesh = plsc.VectorSubcoreMesh(
        core_axis_name="c", subcore_axis_name="s",
        num_cores=NC, num_subcores=NS)
    return pl.kernel(
        _sc_counts_body,
        out_type=jax.ShapeDtypeStruct((NC, NP * NP), jnp.float32),
        mesh=mesh,
        scratch_types=[
            pltpu.VMEM_SHARED((NP * NP,), jnp.float32),
            pltpu.VMEM((CH,), jnp.int32),
            pltpu.VMEM((CH,), jnp.int32),
            pltpu.VMEM((N_STREAMS, IDX_W), jnp.int32),
            pltpu.VMEM((IDX_W,), jnp.float32),
            pltpu.VMEM((ZCH,), jnp.float32),
            pltpu.SemaphoreType.DMA,
        ],
    )(row, col)


def _tc_evolve_body(x_ref, s_ref, wih_ref, whh_ref, bih_ref, bhh_ref,
                    w0_ref, o_ref, xt_s, r_s, z_s):
    g = pl.program_id(0)
    w0 = w0_ref[...]

    @pl.when(g == 0)
    def _():
        xp = jnp.concatenate(
            [x_ref[...], jnp.zeros((NP - N, D), jnp.float32)], axis=0)
        s_col = jnp.concatenate(
            [s_ref[...], jnp.full((NP - N, 1), NEG, jnp.float32)], axis=0)

        ir = lax.broadcasted_iota(jnp.int32, (NP, NP), 0)
        ic = lax.broadcasted_iota(jnp.int32, (NP, NP), 1)
        s_row = lax.transpose(s_col, (1, 0))

        g1 = jnp.broadcast_to(s_col, (NP, NP))
        g2 = jnp.broadcast_to(s_row, (NP, NP))
        beats = (g1 > g2) | ((g1 == g2) & (ir < ic))
        rank_row = jnp.sum(beats.astype(jnp.float32), axis=0, keepdims=True)

        sel_r = lax.broadcasted_iota(jnp.int32, (D, NP), 0)
        rank_i = rank_row.astype(jnp.int32)
        gate = jnp.broadcast_to(jnp.tanh(s_row), (D, NP))
        p_mat = jnp.where(sel_r == jnp.broadcast_to(rank_i, (D, NP)),
                          gate, 0.0)
        xt_s[...] = jnp.dot(p_mat, xp, preferred_element_type=jnp.float32)

    gi_g = lax.dot_general(xt_s[...], wih_ref[...], (((1,), (1,)), ((), ())),
                           preferred_element_type=jnp.float32) + bih_ref[...]
    gh_g = lax.dot_general(w0, whh_ref[...], (((1,), (1,)), ((), ())),
                           preferred_element_type=jnp.float32) + bhh_ref[...]

    @pl.when(g == 0)
    def _():
        r_s[...] = jax.nn.sigmoid(gi_g + gh_g)

    @pl.when(g == 1)
    def _():
        z_s[...] = jax.nn.sigmoid(gi_g + gh_g)

    @pl.when(g == 2)
    def _():
        n = jnp.tanh(gi_g + r_s[...] * gh_g)
        z = z_s[...]
        w_new = (1.0 - z) * n + z * w0
        xp = jnp.concatenate(
            [x_ref[...], jnp.zeros((NP - N, D), jnp.float32)], axis=0)
        o_ref[...] = jnp.dot(xp, w_new, preferred_element_type=jnp.float32)


@jax.jit
def _tc_evolve(x, s_col, w_ih, w_hh, b_ih, b_hh, w0):
    return pl.pallas_call(
        _tc_evolve_body,
        grid=(3,),
        in_specs=[
            pl.BlockSpec((N, D), lambda g: (0, 0)),
            pl.BlockSpec((N, 1), lambda g: (0, 0)),
            pl.BlockSpec((D, D), lambda g: (g, 0)),
            pl.BlockSpec((D, D), lambda g: (g, 0)),
            pl.BlockSpec((1, D), lambda g: (0, g)),
            pl.BlockSpec((1, D), lambda g: (0, g)),
            pl.BlockSpec((D, D), lambda g: (0, 0)),
        ],
        out_specs=pl.BlockSpec((NP, D), lambda g: (0, 0)),
        out_shape=jax.ShapeDtypeStruct((NP, D), jnp.float32),
        scratch_shapes=[
            pltpu.VMEM((D, D), jnp.float32),
            pltpu.VMEM((D, D), jnp.float32),
            pltpu.VMEM((D, D), jnp.float32),
        ],
    )(x, s_col, w_ih, w_hh, b_ih, b_hh, w0)


def _tc_aggregate_body(b_ref, x_ref, o_ref):
    ir = lax.broadcasted_iota(jnp.int32, (NP, 128), 0)
    ic = lax.broadcasted_iota(jnp.int32, (NP, 128), 1)
    nblk = NP // 128
    blocks = []
    deg = jnp.zeros((NP, 1), jnp.float32)
    for b in range(nblk):
        vb = (b_ref[0, b * NP:(b + 1) * NP, :]
              + b_ref[1, b * NP:(b + 1) * NP, :]
              + ((ic + 128 * b) == ir).astype(jnp.float32))
        blocks.append(vb)
        deg = deg + jnp.sum(vb, axis=1, keepdims=True)
    dis = lax.rsqrt(deg)
    xs = x_ref[...] * dis
    y = jnp.dot(blocks[0], xs[0:128, :], preferred_element_type=jnp.float32)
    for b in range(1, nblk):
        y = y + jnp.dot(blocks[b], xs[b * 128:(b + 1) * 128, :],
                        preferred_element_type=jnp.float32)
    o_ref[...] = (y * dis)[:N, :]


@jax.jit
def _tc_aggregate(b2, x):
    return pl.pallas_call(
        _tc_aggregate_body,
        out_shape=jax.ShapeDtypeStruct((N, D), jnp.float32),
    )(b2, x)


def kernel(X, edge_index, p, W_ih, W_hh, b_ih, b_hh, W0):
    npad = EP - edge_index.shape[1]
    row = jnp.concatenate(
        [edge_index[0], jnp.arange(npad, dtype=jnp.int32) % NP])
    col = jnp.concatenate(
        [edge_index[1], jnp.full((npad,), NP - 1, dtype=jnp.int32)])

    score = (X @ p) / (jnp.linalg.norm(p) + 1e-16)

    b2 = _sc_counts(row, col)
    x = _tc_evolve(X, score.reshape(N, 1), W_ih, W_hh,
                   b_ih.reshape(1, 3 * D), b_hh.reshape(1, 3 * D), W0)
    return _tc_aggregate(b2.reshape(NC, NP * NP // 128, 128), x)


# speedup vs baseline: 1.0274x; 1.0274x over previous
import functools

import jax
import jax.numpy as jnp
from jax import lax
from jax.experimental import pallas as pl
from jax.experimental.pallas import tpu as pltpu
from jax.experimental.pallas import tpu_sc as plsc

N = 1271
D = 768
NP = 1280
NC = 2
NS = 16
CH = 3200
EP = NC * NS * CH
IDX_W = 128
N_STREAMS = CH // IDX_W
SLICE = NP * NP // NS
ZCH = 12800
NEG = -3.0e38


def _sc_counts_body(row_hbm, col_hbm, out_hbm,
                    a_sh, row_v, col_v, idx_v, ones_v, zero_v, sem, sem2):
    cid = lax.axis_index("c")
    sid = lax.axis_index("s")

    zeros16 = jnp.zeros((16,), jnp.float32)
    ones16 = jnp.ones((16,), jnp.float32)

    def fill_zero(i, _):
        zero_v[pl.ds(i * 16, 16)] = zeros16
        return 0
    lax.fori_loop(0, ZCH // 16, fill_zero, 0)
    for k in range(IDX_W // 16):
        ones_v[pl.ds(k * 16, 16)] = ones16

    zdescs = [
        pltpu.async_copy(zero_v, a_sh.at[pl.ds(sid * SLICE + k * ZCH, ZCH)],
                         sem2)
        for k in range(SLICE // ZCH)
    ]
    base = cid * (NS * CH) + sid * CH
    edescs = [pltpu.async_copy(row_hbm.at[pl.ds(base, CH)], row_v, sem),
              pltpu.async_copy(col_hbm.at[pl.ds(base, CH)], col_v, sem)]
    for d in edescs:
        d.wait()

    def compute_idx(j, _):
        for k in range(8):
            off = j * IDX_W + k * 16
            r = row_v[pl.ds(off, 16)]
            c = col_v[pl.ds(off, 16)]
            t = r >> 7
            idx_v[j, pl.ds(k * 16, 16)] = (
                (t << 17) + (t << 15) + (c << 7) + (r & 127))
        return 0
    lax.fori_loop(0, N_STREAMS, compute_idx, 0)

    for d in zdescs:
        d.wait()
    plsc.subcore_barrier()

    descs = []
    for j in range(N_STREAMS):
        descs.append(
            pltpu.async_copy(ones_v, a_sh.at[idx_v.at[j]], sem, add=True))
    for d in descs:
        d.wait()
    plsc.subcore_barrier()

    pltpu.sync_copy(a_sh.at[pl.ds(sid * SLICE, SLICE)],
                    out_hbm.at[cid, pl.ds(sid * SLICE, SLICE)])


@jax.jit
def _sc_counts(row, col):
    mesh = plsc.VectorSubcoreMesh(
        core_axis_name="c", subcore_axis_name="s",
        num_cores=NC, num_subcores=NS)
    return pl.kernel(
        _sc_counts_body,
        out_type=jax.ShapeDtypeStruct((NC, NP * NP), jnp.float32),
        mesh=mesh,
        scratch_types=[
            pltpu.VMEM_SHARED((NP * NP,), jnp.float32),
            pltpu.VMEM((CH,), jnp.int32),
            pltpu.VMEM((CH,), jnp.int32),
            pltpu.VMEM((N_STREAMS, IDX_W), jnp.int32),
            pltpu.VMEM((IDX_W,), jnp.float32),
            pltpu.VMEM((ZCH,), jnp.float32),
            pltpu.SemaphoreType.DMA,
            pltpu.SemaphoreType.DMA,
        ],
    )(row, col)


def _tc_evolve_body(x_ref, s_ref, wih_ref, whh_ref, bih_ref, bhh_ref,
                    w0_ref, o_ref, xt_s, r_s, z_s):
    g = pl.program_id(0)
    w0 = w0_ref[...]

    @pl.when(g == 0)
    def _():
        xp = jnp.concatenate(
            [x_ref[...], jnp.zeros((NP - N, D), jnp.float32)], axis=0)
        s_col = jnp.concatenate(
            [s_ref[...], jnp.full((NP - N, 1), NEG, jnp.float32)], axis=0)

        ir = lax.broadcasted_iota(jnp.int32, (NP, NP), 0)
        ic = lax.broadcasted_iota(jnp.int32, (NP, NP), 1)
        s_row = lax.transpose(s_col, (1, 0))

        g1 = jnp.broadcast_to(s_col, (NP, NP))
        g2 = jnp.broadcast_to(s_row, (NP, NP))
        beats = (g1 > g2) | ((g1 == g2) & (ir < ic))
        rank_row = jnp.sum(beats.astype(jnp.float32), axis=0, keepdims=True)

        sel_r = lax.broadcasted_iota(jnp.int32, (D, NP), 0)
        rank_i = rank_row.astype(jnp.int32)
        gate = jnp.broadcast_to(jnp.tanh(s_row), (D, NP))
        p_mat = jnp.where(sel_r == jnp.broadcast_to(rank_i, (D, NP)),
                          gate, 0.0)
        xt_s[...] = jnp.dot(p_mat, xp, preferred_element_type=jnp.float32)

    gi_g = lax.dot_general(xt_s[...], wih_ref[...], (((1,), (1,)), ((), ())),
                           preferred_element_type=jnp.float32) + bih_ref[...]
    gh_g = lax.dot_general(w0, whh_ref[...], (((1,), (1,)), ((), ())),
                           preferred_element_type=jnp.float32) + bhh_ref[...]

    @pl.when(g == 0)
    def _():
        r_s[...] = jax.nn.sigmoid(gi_g + gh_g)

    @pl.when(g == 1)
    def _():
        z_s[...] = jax.nn.sigmoid(gi_g + gh_g)

    @pl.when(g == 2)
    def _():
        n = jnp.tanh(gi_g + r_s[...] * gh_g)
        z = z_s[...]
        w_new = (1.0 - z) * n + z * w0
        xp = jnp.concatenate(
            [x_ref[...], jnp.zeros((NP - N, D), jnp.float32)], axis=0)
        o_ref[...] = jnp.dot(xp, w_new, preferred_element_type=jnp.float32)


@jax.jit
def _tc_evolve(x, s_col, w_ih, w_hh, b_ih, b_hh, w0):
    return pl.pallas_call(
        _tc_evolve_body,
        grid=(3,),
        in_specs=[
            pl.BlockSpec((N, D), lambda g: (0, 0)),
            pl.BlockSpec((N, 1), lambda g: (0, 0)),
            pl.BlockSpec((D, D), lambda g: (g, 0)),
            pl.BlockSpec((D, D), lambda g: (g, 0)),
            pl.BlockSpec((1, D), lambda g: (0, g)),
            pl.BlockSpec((1, D), lambda g: (0, g)),
            pl.BlockSpec((D, D), lambda g: (0, 0)),
        ],
        out_specs=pl.BlockSpec((NP, D), lambda g: (0, 0)),
        out_shape=jax.ShapeDtypeStruct((NP, D), jnp.float32),
        scratch_shapes=[
            pltpu.VMEM((D, D), jnp.float32),
            pltpu.VMEM((D, D), jnp.float32),
            pltpu.VMEM((D, D), jnp.float32),
        ],
    )(x, s_col, w_ih, w_hh, b_ih, b_hh, w0)


def _tc_aggregate_body(b_ref, x_ref, o_ref):
    ir = lax.broadcasted_iota(jnp.int32, (NP, 128), 0)
    ic = lax.broadcasted_iota(jnp.int32, (NP, 128), 1)
    nblk = NP // 128
    blocks = []
    deg = jnp.zeros((NP, 1), jnp.float32)
    for b in range(nblk):
        vb = (b_ref[0, b * NP:(b + 1) * NP, :]
              + b_ref[1, b * NP:(b + 1) * NP, :]
              + ((ic + 128 * b) == ir).astype(jnp.float32))
        blocks.append(vb)
        deg = deg + jnp.sum(vb, axis=1, keepdims=True)
    dis = lax.rsqrt(deg)
    xs = x_ref[...] * dis
    y = jnp.dot(blocks[0], xs[0:128, :], preferred_element_type=jnp.float32)
    for b in range(1, nblk):
        y = y + jnp.dot(blocks[b], xs[b * 128:(b + 1) * 128, :],
                        preferred_element_type=jnp.float32)
    o_ref[...] = (y * dis)[:N, :]


@jax.jit
def _tc_aggregate(b2, x):
    return pl.pallas_call(
        _tc_aggregate_body,
        out_shape=jax.ShapeDtypeStruct((N, D), jnp.float32),
    )(b2, x)


def kernel(X, edge_index, p, W_ih, W_hh, b_ih, b_hh, W0):
    npad = EP - edge_index.shape[1]
    row = jnp.concatenate(
        [edge_index[0], jnp.arange(npad, dtype=jnp.int32) % NP])
    col = jnp.concatenate(
        [edge_index[1], jnp.full((npad,), NP - 1, dtype=jnp.int32)])

    score = (X @ p) / (jnp.linalg.norm(p) + 1e-16)

    b2 = _sc_counts(row, col)
    x = _tc_evolve(X, score.reshape(N, 1), W_ih, W_hh,
                   b_ih.reshape(1, 3 * D), b_hh.reshape(1, 3 * D), W0)
    return _tc_aggregate(b2.reshape(NC, NP * NP // 128, 128), x)
